# baseline (speedup 1.0000x reference)
import functools

import jax
import jax.numpy as jnp
from jax import lax
from jax.experimental import pallas as pl
from jax.experimental.pallas import tpu as pltpu
from jax.experimental.pallas import tpu_sc as plsc

NUM_USERS = 100000
NUM_ITEMS = 50000
LATENT_DIM = 64
BATCH = 4096

_info = plsc.get_sparse_core_info()
_NC, _NS, _NL = _info.num_cores, _info.num_subcores, _info.num_lanes

_HALF = 50048
_REST = NUM_USERS - _HALF
_UBIG = NUM_USERS // 128 * 128 - _HALF
_IBIG = NUM_ITEMS // 128 * 128


def _make_gather_kernel():
    mesh = plsc.VectorSubcoreMesh(core_axis_name="c", subcore_axis_name="s")

    @functools.partial(
        pl.kernel,
        mesh=mesh,
        out_type=[
            jax.ShapeDtypeStruct((LATENT_DIM, BATCH), jnp.float32),
            jax.ShapeDtypeStruct((LATENT_DIM, BATCH), jnp.float32),
        ],
        scratch_types=[
            pltpu.VMEM((2, _HALF), jnp.float32),
            pltpu.VMEM((BATCH,), jnp.int32),
            pltpu.VMEM((BATCH,), jnp.int32),
            pltpu.VMEM((2, BATCH), jnp.float32),
            pltpu.VMEM((2, BATCH), jnp.float32),
            pltpu.SemaphoreType.DMA,
            pltpu.SemaphoreType.DMA,
        ],
        compiler_params=pltpu.CompilerParams(needs_layout_passes=False),
    )
    def gather2(u_t, i_t, u_tail, i_tail, u_idx, i_idx, u_out, i_out,
                colab, u_idx_v, i_idx_v, ob_u, ob_i, sem_in, sem_out):
        cid = lax.axis_index("c")
        sid = lax.axis_index("s")
        col0 = cid * (LATENT_DIM // 2) + sid * 2

        pltpu.sync_copy(u_idx.at[:], u_idx_v)
        pltpu.sync_copy(i_idx.at[:], i_idx_v)

        k0 = jnp.zeros((_NL,), dtype=jnp.int32)
        k1 = jnp.ones((_NL,), dtype=jnp.int32)

        def pass_init(idx_v, ob):
            def body(i, carry):
                sl = pl.ds(i * _NL, _NL)
                iv = idx_v[sl]
                m = iv < _HALF
                ob[0, sl] = plsc.load_gather(colab, [k0, iv], mask=m)
                ob[1, sl] = plsc.load_gather(colab, [k1, iv], mask=m)
                return carry
            lax.fori_loop(0, BATCH // _NL, body, 0, unroll=8)

        def pass_merge(idx_v, ob):
            def body(i, carry):
                sl = pl.ds(i * _NL, _NL)
                loc = idx_v[sl] - _HALF
                m = loc >= 0
                ob[0, sl] = ob[0, sl] + plsc.load_gather(
                    colab, [k0, loc], mask=m)
                ob[1, sl] = ob[1, sl] + plsc.load_gather(
                    colab, [k1, loc], mask=m)
                return carry
            lax.fori_loop(0, BATCH // _NL, body, 0, unroll=8)

        def pass_plain(idx_v, ob):
            def body(i, carry):
                sl = pl.ds(i * _NL, _NL)
                iv = idx_v[sl]
                ob[0, sl] = plsc.load_gather(colab, [k0, iv])
                ob[1, sl] = plsc.load_gather(colab, [k1, iv])
                return carry
            lax.fori_loop(0, BATCH // _NL, body, 0, unroll=8)

        c1 = pltpu.async_copy(
            u_t.at[pl.ds(col0, 2), pl.ds(0, _HALF)], colab, sem_in)
        c1.wait()
        pass_init(u_idx_v, ob_u)
        c2 = pltpu.async_copy(
            u_t.at[pl.ds(col0, 2), pl.ds(_HALF, _UBIG)],
            colab.at[:, pl.ds(0, _UBIG)], sem_in)
        c2t = pltpu.async_copy(
            u_tail.at[pl.ds(col0, 2), :],
            colab.at[:, pl.ds(_UBIG, 128)], sem_in)
        c2.wait()
        c2t.wait()
        pass_merge(u_idx_v, ob_u)
        o1 = pltpu.async_copy(ob_u, u_out.at[pl.ds(col0, 2), :], sem_out)
        c3 = pltpu.async_copy(
            i_t.at[pl.ds(col0, 2), pl.ds(0, _IBIG)],
            colab.at[:, pl.ds(0, _IBIG)], sem_in)
        c3t = pltpu.async_copy(
            i_tail.at[pl.ds(col0, 2), :],
            colab.at[:, pl.ds(_IBIG, 128)], sem_in)
        c3.wait()
        c3t.wait()
        pass_plain(i_idx_v, ob_i)
        o2 = pltpu.async_copy(ob_i, i_out.at[pl.ds(col0, 2), :], sem_out)
        o1.wait()
        o2.wait()

    return gather2


_gather2 = _make_gather_kernel()


def kernel(user_indices, item_indices, user_embeddings, item_embeddings):
    u_idx = user_indices.astype(jnp.int32)
    i_idx = item_indices.astype(jnp.int32)
    u_tail = jnp.pad(user_embeddings[NUM_USERS // 128 * 128:, :],
                     ((0, 128 - NUM_USERS % 128), (0, 0))).T
    i_tail = jnp.pad(item_embeddings[NUM_ITEMS // 128 * 128:, :],
                     ((0, 128 - NUM_ITEMS % 128), (0, 0))).T
    u_out_t, i_out_t = _gather2(user_embeddings.T, item_embeddings.T,
                                u_tail, i_tail, u_idx, i_idx)
    return (u_out_t.T, i_out_t.T)


# speedup vs baseline: 1.0226x; 1.0226x over previous
import functools

import jax
import jax.numpy as jnp
from jax import lax
from jax.experimental import pallas as pl
from jax.experimental.pallas import tpu as pltpu
from jax.experimental.pallas import tpu_sc as plsc

NUM_USERS = 100000
NUM_ITEMS = 50000
LATENT_DIM = 64
BATCH = 4096

_info = plsc.get_sparse_core_info()
_NC, _NS, _NL = _info.num_cores, _info.num_subcores, _info.num_lanes

_SPAN = 26496
_UBULK = NUM_USERS // 128 * 128
_IBULK = NUM_ITEMS // 128 * 128
_USTAGES = [(0, _SPAN), (_SPAN, _SPAN), (2 * _SPAN, _SPAN),
            (3 * _SPAN, _UBULK + 128 - 3 * _SPAN)]
_ISTAGES = [(0, _SPAN), (_SPAN, _IBULK + 128 - _SPAN)]


def _make_gather_kernel():
    mesh = plsc.VectorSubcoreMesh(core_axis_name="c", subcore_axis_name="s")

    @functools.partial(
        pl.kernel,
        mesh=mesh,
        out_type=[
            jax.ShapeDtypeStruct((LATENT_DIM, BATCH), jnp.float32),
            jax.ShapeDtypeStruct((LATENT_DIM, BATCH), jnp.float32),
        ],
        scratch_types=[
            pltpu.VMEM((2, _SPAN), jnp.float32),
            pltpu.VMEM((2, _SPAN), jnp.float32),
            pltpu.VMEM((BATCH,), jnp.int32),
            pltpu.VMEM((BATCH,), jnp.int32),
            pltpu.VMEM((2, BATCH), jnp.float32),
            pltpu.VMEM((2, BATCH), jnp.float32),
            pltpu.SemaphoreType.DMA,
            pltpu.SemaphoreType.DMA,
            pltpu.SemaphoreType.DMA,
        ],
        compiler_params=pltpu.CompilerParams(needs_layout_passes=False),
    )
    def gather2(u_t, i_t, u_tail, i_tail, u_idx, i_idx, u_out, i_out,
                buf_a, buf_b, u_idx_v, i_idx_v, ob_u, ob_i,
                sem_a, sem_b, sem_out):
        cid = lax.axis_index("c")
        sid = lax.axis_index("s")
        col0 = cid * (LATENT_DIM // 2) + sid * 2

        pltpu.sync_copy(u_idx.at[:], u_idx_v)
        pltpu.sync_copy(i_idx.at[:], i_idx_v)

        k0 = jnp.zeros((_NL,), dtype=jnp.int32)
        k1 = jnp.ones((_NL,), dtype=jnp.int32)

        def fire(table, tail, bulk_end, stage, buf, sem):
            s0, ln = stage
            if s0 + ln <= bulk_end:
                return [pltpu.async_copy(
                    table.at[pl.ds(col0, 2), pl.ds(s0, ln)],
                    buf.at[:, pl.ds(0, ln)], sem)]
            return [
                pltpu.async_copy(
                    table.at[pl.ds(col0, 2), pl.ds(s0, ln - 128)],
                    buf.at[:, pl.ds(0, ln - 128)], sem),
                pltpu.async_copy(
                    tail.at[pl.ds(col0, 2), :],
                    buf.at[:, pl.ds(ln - 128, 128)], sem),
            ]

        def gpass(idx_v, ob, buf, stage, first, last):
            s0, ln = stage

            def body(i, carry):
                sl = pl.ds(i * _NL, _NL)
                loc = idx_v[sl] - s0
                if first:
                    m = loc < ln
                elif last:
                    m = loc >= 0
                else:
                    m = (loc >= 0) & (loc < ln)
                v0 = plsc.load_gather(buf, [k0, loc], mask=m)
                v1 = plsc.load_gather(buf, [k1, loc], mask=m)
                if first:
                    ob[0, sl] = v0
                    ob[1, sl] = v1
                else:
                    ob[0, sl] = ob[0, sl] + v0
                    ob[1, sl] = ob[1, sl] + v1
                return carry
            lax.fori_loop(0, BATCH // _NL, body, 0, unroll=8)

        def wait(handles):
            for h in handles:
                h.wait()

        hu1 = fire(u_t, u_tail, _UBULK, _USTAGES[0], buf_a, sem_a)
        hu2 = fire(u_t, u_tail, _UBULK, _USTAGES[1], buf_b, sem_b)
        wait(hu1)
        gpass(u_idx_v, ob_u, buf_a, _USTAGES[0], True, False)
        hu3 = fire(u_t, u_tail, _UBULK, _USTAGES[2], buf_a, sem_a)
        wait(hu2)
        gpass(u_idx_v, ob_u, buf_b, _USTAGES[1], False, False)
        hu4 = fire(u_t, u_tail, _UBULK, _USTAGES[3], buf_b, sem_b)
        wait(hu3)
        gpass(u_idx_v, ob_u, buf_a, _USTAGES[2], False, False)
        hi1 = fire(i_t, i_tail, _IBULK, _ISTAGES[0], buf_a, sem_a)
        wait(hu4)
        gpass(u_idx_v, ob_u, buf_b, _USTAGES[3], False, True)
        o1 = pltpu.async_copy(ob_u, u_out.at[pl.ds(col0, 2), :], sem_out)
        hi2 = fire(i_t, i_tail, _IBULK, _ISTAGES[1], buf_b, sem_b)
        wait(hi1)
        gpass(i_idx_v, ob_i, buf_a, _ISTAGES[0], True, False)
        wait(hi2)
        gpass(i_idx_v, ob_i, buf_b, _ISTAGES[1], False, True)
        o2 = pltpu.async_copy(ob_i, i_out.at[pl.ds(col0, 2), :], sem_out)
        o1.wait()
        o2.wait()

    return gather2


_gather2 = _make_gather_kernel()


def kernel(user_indices, item_indices, user_embeddings, item_embeddings):
    u_idx = user_indices.astype(jnp.int32)
    i_idx = item_indices.astype(jnp.int32)
    u_tail = jnp.pad(user_embeddings[NUM_USERS // 128 * 128:, :],
                     ((0, 128 - NUM_USERS % 128), (0, 0))).T
    i_tail = jnp.pad(item_embeddings[NUM_ITEMS // 128 * 128:, :],
                     ((0, 128 - NUM_ITEMS % 128), (0, 0))).T
    u_out_t, i_out_t = _gather2(user_embeddings.T, item_embeddings.T,
                                u_tail, i_tail, u_idx, i_idx)
    return (u_out_t.T, i_out_t.T)


# speedup vs baseline: 1.0600x; 1.0365x over previous
import functools

import jax
import jax.numpy as jnp
from jax import lax
from jax.experimental import pallas as pl
from jax.experimental.pallas import tpu as pltpu
from jax.experimental.pallas import tpu_sc as plsc

NUM_USERS = 100000
NUM_ITEMS = 50000
LATENT_DIM = 64
BATCH = 4096

_info = plsc.get_sparse_core_info()
_NC, _NS, _NL = _info.num_cores, _info.num_subcores, _info.num_lanes

_SPAN = 26496
_UBULK = NUM_USERS // 128 * 128
_IBULK = NUM_ITEMS // 128 * 128
_USTAGES = [(0, _SPAN), (_SPAN, _SPAN), (2 * _SPAN, _SPAN),
            (3 * _SPAN, _UBULK + 128 - 3 * _SPAN)]
_ISTAGES = [(0, _SPAN), (_SPAN, _IBULK + 128 - _SPAN)]


def _make_gather_kernel():
    mesh = plsc.VectorSubcoreMesh(core_axis_name="c", subcore_axis_name="s")

    @functools.partial(
        pl.kernel,
        mesh=mesh,
        out_type=[
            jax.ShapeDtypeStruct((LATENT_DIM, BATCH), jnp.float32),
            jax.ShapeDtypeStruct((LATENT_DIM, BATCH), jnp.float32),
        ],
        scratch_types=[
            pltpu.VMEM((2, _SPAN), jnp.float32),
            pltpu.VMEM((2, _SPAN), jnp.float32),
            pltpu.VMEM((BATCH,), jnp.int32),
            pltpu.VMEM((BATCH,), jnp.int32),
            pltpu.VMEM((2, BATCH), jnp.float32),
            pltpu.VMEM((2, BATCH), jnp.float32),
            pltpu.SemaphoreType.DMA,
            pltpu.SemaphoreType.DMA,
            pltpu.SemaphoreType.DMA,
        ],
        compiler_params=pltpu.CompilerParams(needs_layout_passes=False),
    )
    def gather2(u_t, i_t, u_tail, i_tail, u_idx, i_idx, u_out, i_out,
                buf_a, buf_b, u_idx_v, i_idx_v, ob_u, ob_i,
                sem_a, sem_b, sem_out):
        cid = lax.axis_index("c")
        sid = lax.axis_index("s")
        col0 = cid * (LATENT_DIM // 2) + sid * 2

        k0 = jnp.zeros((_NL,), dtype=jnp.int32)
        k1 = jnp.ones((_NL,), dtype=jnp.int32)

        def fire(table, tail, bulk_end, stage, buf, sem):
            s0, ln = stage
            if s0 + ln <= bulk_end:
                return [pltpu.async_copy(
                    table.at[pl.ds(col0, 2), pl.ds(s0, ln)],
                    buf.at[:, pl.ds(0, ln)], sem)]
            return [
                pltpu.async_copy(
                    table.at[pl.ds(col0, 2), pl.ds(s0, ln - 128)],
                    buf.at[:, pl.ds(0, ln - 128)], sem),
                pltpu.async_copy(
                    tail.at[pl.ds(col0, 2), :],
                    buf.at[:, pl.ds(ln - 128, 128)], sem),
            ]

        def gpass(idx_v, ob, buf, stage, first, last):
            s0, ln = stage

            def body(i, carry):
                sl = pl.ds(i * _NL, _NL)
                loc = idx_v[sl] - s0
                if first:
                    m = loc < ln
                elif last:
                    m = loc >= 0
                else:
                    m = (loc >= 0) & (loc < ln)
                v0 = plsc.load_gather(buf, [k0, loc], mask=m)
                v1 = plsc.load_gather(buf, [k1, loc], mask=m)
                if first:
                    ob[0, sl] = v0
                    ob[1, sl] = v1
                else:
                    ob[0, sl] = ob[0, sl] + v0
                    ob[1, sl] = ob[1, sl] + v1
                return carry
            lax.fori_loop(0, BATCH // _NL, body, 0, unroll=8)

        def wait(handles):
            for h in handles:
                h.wait()

        hu1 = fire(u_t, u_tail, _UBULK, _USTAGES[0], buf_a, sem_a)
        hu2 = fire(u_t, u_tail, _UBULK, _USTAGES[1], buf_b, sem_b)
        pltpu.sync_copy(u_idx.at[:], u_idx_v)
        pltpu.sync_copy(i_idx.at[:], i_idx_v)
        wait(hu1)
        gpass(u_idx_v, ob_u, buf_a, _USTAGES[0], True, False)
        hu3 = fire(u_t, u_tail, _UBULK, _USTAGES[2], buf_a, sem_a)
        wait(hu2)
        gpass(u_idx_v, ob_u, buf_b, _USTAGES[1], False, False)
        hu4 = fire(u_t, u_tail, _UBULK, _USTAGES[3], buf_b, sem_b)
        wait(hu3)
        gpass(u_idx_v, ob_u, buf_a, _USTAGES[2], False, False)
        hi1 = fire(i_t, i_tail, _IBULK, _ISTAGES[0], buf_a, sem_a)
        wait(hu4)
        gpass(u_idx_v, ob_u, buf_b, _USTAGES[3], False, True)
        o1 = pltpu.async_copy(ob_u, u_out.at[pl.ds(col0, 2), :], sem_out)
        hi2 = fire(i_t, i_tail, _IBULK, _ISTAGES[1], buf_b, sem_b)
        wait(hi1)
        gpass(i_idx_v, ob_i, buf_a, _ISTAGES[0], True, False)
        wait(hi2)
        gpass(i_idx_v, ob_i, buf_b, _ISTAGES[1], False, True)
        o2 = pltpu.async_copy(ob_i, i_out.at[pl.ds(col0, 2), :], sem_out)
        o1.wait()
        o2.wait()

    return gather2


_gather2 = _make_gather_kernel()


def kernel(user_indices, item_indices, user_embeddings, item_embeddings):
    u_idx = user_indices.astype(jnp.int32)
    i_idx = item_indices.astype(jnp.int32)
    u_tail = jnp.pad(user_embeddings[NUM_USERS // 128 * 128:, :],
                     ((0, 128 - NUM_USERS % 128), (0, 0))).T
    i_tail = jnp.pad(item_embeddings[NUM_ITEMS // 128 * 128:, :],
                     ((0, 128 - NUM_ITEMS % 128), (0, 0))).T
    u_out_t, i_out_t = _gather2(user_embeddings.T, item_embeddings.T,
                                u_tail, i_tail, u_idx, i_idx)
    return (u_out_t.T, i_out_t.T)
